# baseline (speedup 1.0000x reference)
import functools

import jax
import jax.numpy as jnp
from jax import lax
from jax.experimental import pallas as pl
from jax.experimental.pallas import tpu as pltpu
from jax.experimental.pallas import tpu_sc as plsc

ALPHA = 2.0
NEG_WEIGHT = 1.0
MIN_IOU = 0.05

L = 16
ROWS = 16
B, M, N = 32, 1024, 1024
NCHUNK = M // ROWS
NJC = N // L


def _body(scores_hbm, ious_hbm, lastcol_hbm, lastrow_hbm, out_hbm,
          iou_buf, sc_buf, lastcol, lastrow, colsum, out_buf, sem):
    cid = lax.axis_index("c")
    sid = lax.axis_index("s")
    b = sid * 2 + cid

    pltpu.sync_copy(lastcol_hbm.at[b], lastcol)
    pltpu.sync_copy(lastrow_hbm.at[b], lastrow)

    def zbody(j, _):
        colsum[pl.ds(j * L, L)] = jnp.zeros((L,), jnp.float32)
        return 0
    lax.fori_loop(0, NJC, zbody, 0)

    zero_v = jnp.zeros((L,), jnp.float32)

    def chunk_body(g, carry):
        s1, cnt, n0c, n0d = carry
        r0 = g * ROWS
        pltpu.sync_copy(ious_hbm.at[b, pl.ds(r0, ROWS), pl.ds(0, N)], iou_buf)
        pltpu.sync_copy(scores_hbm.at[b, pl.ds(r0, ROWS), pl.ds(0, N)], sc_buf)

        def jc_body(jc, c):
            s1_, cnt_, raccs = c
            off = jc * L
            colacc = colsum[pl.ds(off, L)]
            new_raccs = []
            for i in range(ROWS):
                vio = iou_buf[i, pl.ds(off, L)]
                vsc = sc_buf[i, pl.ds(off, L)]
                m = vio >= MIN_IOU
                t = jnp.where(m, jnp.minimum(vio, 1.0), 0.0)
                s1_ = s1_ + vsc * t
                cnt_ = cnt_ + jnp.where(m, 1.0, 0.0)
                colacc = colacc + t
                new_raccs.append(raccs[i] + t)
            colsum[pl.ds(off, L)] = colacc
            return s1_, cnt_, tuple(new_raccs)

        s1, cnt, raccs = lax.fori_loop(
            0, NJC, jc_body, (s1, cnt, (zero_v,) * ROWS))

        lc = lastcol[pl.ds(r0, L)]
        for i in range(ROWS):
            rs = jnp.sum(raccs[i])
            f = jnp.where(rs <= 0.001, 1.0, 0.0)
            n0c = n0c + f
            n0d = n0d + f * lc[i]
        return s1, cnt, n0c, n0d

    s1, cnt, n0c, n0d = lax.fori_loop(
        0, NCHUNK, chunk_body,
        (zero_v, zero_v, jnp.float32(0.0), jnp.float32(0.0)))

    def neg1_body(jc, c):
        n1c_, n1d_ = c
        off = jc * L
        v = colsum[pl.ds(off, L)]
        lr = lastrow[pl.ds(off, L)]
        n1 = jnp.where(v <= 0.001, 1.0, 0.0)
        return n1c_ + n1, n1d_ + n1 * lr
    n1c, n1d = lax.fori_loop(0, NJC, neg1_body, (zero_v, zero_v))

    s1s = jnp.full((L,), jnp.sum(s1), jnp.float32)
    cnts = jnp.full((L,), jnp.sum(cnt), jnp.float32)
    n1cs = jnp.full((L,), jnp.sum(n1c), jnp.float32)
    n1ds = jnp.full((L,), jnp.sum(n1d), jnp.float32)
    n0cv = jnp.full((L,), n0c, jnp.float32)
    n0dv = jnp.full((L,), n0d, jnp.float32)

    nll_pos = -(ALPHA * s1s) / jnp.maximum(cnts, 1.0)
    nll_neg = (-n0dv - n1ds) / (jnp.maximum(n0cv, 1.0) + jnp.maximum(n1cs, 1.0))
    loss = (nll_pos + NEG_WEIGHT * nll_neg) * (1.0 / B)

    out_buf[...] = loss
    pltpu.sync_copy(out_buf, out_hbm.at[b])


@jax.jit
def _run(scores, ious):
    lastcol = scores[:, :M, N]
    lastrow = scores[:, M, :N]
    mesh = plsc.VectorSubcoreMesh(core_axis_name="c", subcore_axis_name="s")
    f = pl.kernel(
        _body,
        out_type=jax.ShapeDtypeStruct((B, L), jnp.float32),
        mesh=mesh,
        scratch_types=[
            pltpu.VMEM((ROWS, N), jnp.float32),
            pltpu.VMEM((ROWS, N), jnp.float32),
            pltpu.VMEM((M,), jnp.float32),
            pltpu.VMEM((N,), jnp.float32),
            pltpu.VMEM((N,), jnp.float32),
            pltpu.VMEM((L,), jnp.float32),
            pltpu.SemaphoreType.DMA,
        ],
        compiler_params=pltpu.CompilerParams(
            use_tc_tiling_on_sc=False, needs_layout_passes=False),
    )
    per_scene = f(scores, ious, lastcol, lastrow)
    return jnp.sum(per_scene[:, 0])


def kernel(logmax_scores, instance_ious, instance_matches):
    del instance_matches
    return _run(logmax_scores, instance_ious)


# speedup vs baseline: 1.0808x; 1.0808x over previous
import functools

import jax
import jax.numpy as jnp
from jax import lax
from jax.experimental import pallas as pl
from jax.experimental.pallas import tpu as pltpu
from jax.experimental.pallas import tpu_sc as plsc

ALPHA = 2.0
NEG_WEIGHT = 1.0
MIN_IOU = 0.05

L = 16
ROWS = 16
B, M, N = 32, 1024, 1024
NCHUNK = M // ROWS
NJC = N // L
NPAIR = NCHUNK // 2


def _body(scores_hbm, ious_hbm, lastcol_hbm, lastrow_hbm, out_hbm,
          iou0, iou1, sc0, sc1, lastcol, lastrow, colsum, rowvec, out_buf,
          sem_i0, sem_i1, sem_s0, sem_s1):
    cid = lax.axis_index("c")
    sid = lax.axis_index("s")
    b = sid * 2 + cid

    def iou_copy(g, buf, sem):
        return pltpu.make_async_copy(
            ious_hbm.at[b, pl.ds(g * ROWS, ROWS), pl.ds(0, N)], buf, sem)

    def sc_copy(g, buf, sem):
        return pltpu.make_async_copy(
            scores_hbm.at[b, pl.ds(g * ROWS, ROWS), pl.ds(0, N)], buf, sem)

    iou_copy(0, iou0, sem_i0).start()
    sc_copy(0, sc0, sem_s0).start()
    iou_copy(1, iou1, sem_i1).start()
    sc_copy(1, sc1, sem_s1).start()

    pltpu.sync_copy(lastcol_hbm.at[b], lastcol)
    pltpu.sync_copy(lastrow_hbm.at[b], lastrow)

    def zbody(j, _):
        colsum[pl.ds(j * L, L)] = jnp.zeros((L,), jnp.float32)
        return 0
    lax.fori_loop(0, NJC, zbody, 0)

    zero_v = jnp.zeros((L,), jnp.float32)
    iota16 = jnp.arange(L, dtype=jnp.int32)

    def chunk_compute(g, iou_buf, sc_buf, s1, cnt, n0c, n0d):
        def jc_body(jc, c):
            s1_, cnt_, raccs = c
            off = jc * L
            colacc = colsum[pl.ds(off, L)]
            new_raccs = []
            for i in range(ROWS):
                vio = iou_buf[i, pl.ds(off, L)]
                vsc = sc_buf[i, pl.ds(off, L)]
                m = vio >= MIN_IOU
                t = jnp.where(m, jnp.minimum(vio, 1.0), 0.0)
                s1_ = s1_ + vsc * t
                cnt_ = cnt_ + jnp.where(m, 1.0, 0.0)
                colacc = colacc + t
                new_raccs.append(raccs[i] + t)
            colsum[pl.ds(off, L)] = colacc
            return s1_, cnt_, tuple(new_raccs)

        s1, cnt, raccs = lax.fori_loop(
            0, NJC, jc_body, (s1, cnt, (zero_v,) * ROWS))

        for i in range(ROWS):
            rowvec[i] = raccs[i]
        rowsum = plsc.load_gather(
            rowvec, [iota16, jnp.full((L,), 0, jnp.int32)])
        for c in range(1, L):
            rowsum = rowsum + plsc.load_gather(
                rowvec, [iota16, jnp.full((L,), c, jnp.int32)])
        fvec = jnp.where(rowsum <= 0.001, 1.0, 0.0)
        lc = lastcol[pl.ds(g * ROWS, L)]
        return s1, cnt, n0c + fvec, n0d + fvec * lc

    def pair_body(p, carry):
        s1, cnt, n0c, n0d = carry
        g0 = 2 * p
        iou_copy(g0, iou0, sem_i0).wait()
        sc_copy(g0, sc0, sem_s0).wait()
        s1, cnt, n0c, n0d = chunk_compute(g0, iou0, sc0, s1, cnt, n0c, n0d)
        ge = jnp.minimum(g0 + 2, NCHUNK - 1)
        iou_copy(ge, iou0, sem_i0).start()
        sc_copy(ge, sc0, sem_s0).start()

        g1 = 2 * p + 1
        iou_copy(g1, iou1, sem_i1).wait()
        sc_copy(g1, sc1, sem_s1).wait()
        s1, cnt, n0c, n0d = chunk_compute(g1, iou1, sc1, s1, cnt, n0c, n0d)
        go = jnp.minimum(g1 + 2, NCHUNK - 1)
        iou_copy(go, iou1, sem_i1).start()
        sc_copy(go, sc1, sem_s1).start()
        return s1, cnt, n0c, n0d

    s1, cnt, n0c, n0d = lax.fori_loop(
        0, NPAIR, pair_body, (zero_v, zero_v, zero_v, zero_v))

    iou_copy(NCHUNK - 1, iou0, sem_i0).wait()
    sc_copy(NCHUNK - 1, sc0, sem_s0).wait()
    iou_copy(NCHUNK - 1, iou1, sem_i1).wait()
    sc_copy(NCHUNK - 1, sc1, sem_s1).wait()

    def neg1_body(jc, c):
        n1c_, n1d_ = c
        off = jc * L
        v = colsum[pl.ds(off, L)]
        lr = lastrow[pl.ds(off, L)]
        n1 = jnp.where(v <= 0.001, 1.0, 0.0)
        return n1c_ + n1, n1d_ + n1 * lr
    n1c, n1d = lax.fori_loop(0, NJC, neg1_body, (zero_v, zero_v))

    s1s = jnp.full((L,), jnp.sum(s1), jnp.float32)
    cnts = jnp.full((L,), jnp.sum(cnt), jnp.float32)
    n0cs = jnp.full((L,), jnp.sum(n0c), jnp.float32)
    n0ds = jnp.full((L,), jnp.sum(n0d), jnp.float32)
    n1cs = jnp.full((L,), jnp.sum(n1c), jnp.float32)
    n1ds = jnp.full((L,), jnp.sum(n1d), jnp.float32)

    nll_pos = -(ALPHA * s1s) / jnp.maximum(cnts, 1.0)
    nll_neg = (-n0ds - n1ds) / (jnp.maximum(n0cs, 1.0) + jnp.maximum(n1cs, 1.0))
    loss = (nll_pos + NEG_WEIGHT * nll_neg) * (1.0 / B)

    out_buf[...] = loss
    pltpu.sync_copy(out_buf, out_hbm.at[b])


@jax.jit
def _run(scores, ious):
    lastcol = scores[:, :M, N]
    lastrow = scores[:, M, :N]
    mesh = plsc.VectorSubcoreMesh(core_axis_name="c", subcore_axis_name="s")
    f = pl.kernel(
        _body,
        out_type=jax.ShapeDtypeStruct((B, L), jnp.float32),
        mesh=mesh,
        scratch_types=[
            pltpu.VMEM((ROWS, N), jnp.float32),
            pltpu.VMEM((ROWS, N), jnp.float32),
            pltpu.VMEM((ROWS, N), jnp.float32),
            pltpu.VMEM((ROWS, N), jnp.float32),
            pltpu.VMEM((M,), jnp.float32),
            pltpu.VMEM((N,), jnp.float32),
            pltpu.VMEM((N,), jnp.float32),
            pltpu.VMEM((ROWS, L), jnp.float32),
            pltpu.VMEM((L,), jnp.float32),
            pltpu.SemaphoreType.DMA,
            pltpu.SemaphoreType.DMA,
            pltpu.SemaphoreType.DMA,
            pltpu.SemaphoreType.DMA,
        ],
        compiler_params=pltpu.CompilerParams(
            use_tc_tiling_on_sc=False, needs_layout_passes=False),
    )
    per_scene = f(scores, ious, lastcol, lastrow)
    return jnp.sum(per_scene[:, 0])


def kernel(logmax_scores, instance_ious, instance_matches):
    del instance_matches
    return _run(logmax_scores, instance_ious)


# speedup vs baseline: 7.8285x; 7.2431x over previous
import functools

import jax
import jax.numpy as jnp
from jax import lax
from jax.experimental import pallas as pl
from jax.experimental.pallas import tpu as pltpu
from jax.experimental.pallas import tpu_sc as plsc

ALPHA = 2.0
NEG_WEIGHT = 1.0
MIN_IOU = 0.05

L = 16
ROWS = 16
B, M, N = 32, 1024, 1024
NCHUNK = M // ROWS
NJC = N // L
NPAIR = NCHUNK // 2


def _body(scores_hbm, ious_hbm, out_hbm,
          iou0, iou1, sc0, sc1, lastrow, colsum, rowvec, out_buf,
          sem_i0, sem_i1, sem_s0, sem_s1):
    cid = lax.axis_index("c")
    sid = lax.axis_index("s")
    b = sid * 2 + cid

    def iou_copy(g, buf, sem):
        return pltpu.make_async_copy(
            ious_hbm.at[b, pl.ds(g * ROWS, ROWS), pl.ds(0, N)], buf, sem)

    def sc_copy(g, buf, sem):
        return pltpu.make_async_copy(
            scores_hbm.at[b, pl.ds(g * ROWS, ROWS)], buf, sem)

    iou_copy(0, iou0, sem_i0).start()
    sc_copy(0, sc0, sem_s0).start()
    iou_copy(1, iou1, sem_i1).start()
    sc_copy(1, sc1, sem_s1).start()

    pltpu.sync_copy(scores_hbm.at[b, pl.ds(M, 1)], lastrow)

    def zbody(j, _):
        colsum[pl.ds(j * L, L)] = jnp.zeros((L,), jnp.float32)
        return 0
    lax.fori_loop(0, NJC, zbody, 0)

    zero_v = jnp.zeros((L,), jnp.float32)
    iota16 = jnp.arange(L, dtype=jnp.int32)

    def chunk_compute(g, iou_buf, sc_buf, s1, cnt, n0c, n0d):
        def jc_body(jc, c):
            s1_, cnt_, raccs = c
            off = jc * L
            colacc = colsum[pl.ds(off, L)]
            new_raccs = []
            for i in range(ROWS):
                vio = iou_buf[i, pl.ds(off, L)]
                vsc = sc_buf[i, pl.ds(off, L)]
                m = vio >= MIN_IOU
                t = jnp.where(m, jnp.minimum(vio, 1.0), 0.0)
                s1_ = s1_ + vsc * t
                cnt_ = cnt_ + jnp.where(m, 1.0, 0.0)
                colacc = colacc + t
                new_raccs.append(raccs[i] + t)
            colsum[pl.ds(off, L)] = colacc
            return s1_, cnt_, tuple(new_raccs)

        s1, cnt, raccs = lax.fori_loop(
            0, NJC, jc_body, (s1, cnt, (zero_v,) * ROWS))

        for i in range(ROWS):
            rowvec[i] = raccs[i]
        rowsum = plsc.load_gather(
            rowvec, [iota16, jnp.full((L,), 0, jnp.int32)])
        for c in range(1, L):
            rowsum = rowsum + plsc.load_gather(
                rowvec, [iota16, jnp.full((L,), c, jnp.int32)])
        fvec = jnp.where(rowsum <= 0.001, 1.0, 0.0)
        lc = plsc.load_gather(sc_buf, [iota16, jnp.full((L,), N, jnp.int32)])
        return s1, cnt, n0c + fvec, n0d + fvec * lc

    def pair_body(p, carry):
        s1, cnt, n0c, n0d = carry
        g0 = 2 * p
        iou_copy(g0, iou0, sem_i0).wait()
        sc_copy(g0, sc0, sem_s0).wait()
        s1, cnt, n0c, n0d = chunk_compute(g0, iou0, sc0, s1, cnt, n0c, n0d)
        ge = jnp.minimum(g0 + 2, NCHUNK - 1)
        iou_copy(ge, iou0, sem_i0).start()
        sc_copy(ge, sc0, sem_s0).start()

        g1 = 2 * p + 1
        iou_copy(g1, iou1, sem_i1).wait()
        sc_copy(g1, sc1, sem_s1).wait()
        s1, cnt, n0c, n0d = chunk_compute(g1, iou1, sc1, s1, cnt, n0c, n0d)
        go = jnp.minimum(g1 + 2, NCHUNK - 1)
        iou_copy(go, iou1, sem_i1).start()
        sc_copy(go, sc1, sem_s1).start()
        return s1, cnt, n0c, n0d

    s1, cnt, n0c, n0d = lax.fori_loop(
        0, NPAIR, pair_body, (zero_v, zero_v, zero_v, zero_v))

    iou_copy(NCHUNK - 1, iou0, sem_i0).wait()
    sc_copy(NCHUNK - 1, sc0, sem_s0).wait()
    iou_copy(NCHUNK - 1, iou1, sem_i1).wait()
    sc_copy(NCHUNK - 1, sc1, sem_s1).wait()

    def neg1_body(jc, c):
        n1c_, n1d_ = c
        off = jc * L
        v = colsum[pl.ds(off, L)]
        lr = lastrow[0, pl.ds(off, L)]
        n1 = jnp.where(v <= 0.001, 1.0, 0.0)
        return n1c_ + n1, n1d_ + n1 * lr
    n1c, n1d = lax.fori_loop(0, NJC, neg1_body, (zero_v, zero_v))

    s1s = jnp.full((L,), jnp.sum(s1), jnp.float32)
    cnts = jnp.full((L,), jnp.sum(cnt), jnp.float32)
    n0cs = jnp.full((L,), jnp.sum(n0c), jnp.float32)
    n0ds = jnp.full((L,), jnp.sum(n0d), jnp.float32)
    n1cs = jnp.full((L,), jnp.sum(n1c), jnp.float32)
    n1ds = jnp.full((L,), jnp.sum(n1d), jnp.float32)

    nll_pos = -(ALPHA * s1s) / jnp.maximum(cnts, 1.0)
    nll_neg = (-n0ds - n1ds) / (jnp.maximum(n0cs, 1.0) + jnp.maximum(n1cs, 1.0))
    loss = (nll_pos + NEG_WEIGHT * nll_neg) * (1.0 / B)

    out_buf[...] = loss
    pltpu.sync_copy(out_buf, out_hbm.at[b])


@jax.jit
def _run(scores, ious):
    mesh = plsc.VectorSubcoreMesh(core_axis_name="c", subcore_axis_name="s")
    f = pl.kernel(
        _body,
        out_type=jax.ShapeDtypeStruct((B, L), jnp.float32),
        mesh=mesh,
        scratch_types=[
            pltpu.VMEM((ROWS, N), jnp.float32),
            pltpu.VMEM((ROWS, N), jnp.float32),
            pltpu.VMEM((ROWS, N + 1), jnp.float32),
            pltpu.VMEM((ROWS, N + 1), jnp.float32),
            pltpu.VMEM((1, N + 1), jnp.float32),
            pltpu.VMEM((N,), jnp.float32),
            pltpu.VMEM((ROWS, L), jnp.float32),
            pltpu.VMEM((L,), jnp.float32),
            pltpu.SemaphoreType.DMA,
            pltpu.SemaphoreType.DMA,
            pltpu.SemaphoreType.DMA,
            pltpu.SemaphoreType.DMA,
        ],
        compiler_params=pltpu.CompilerParams(
            use_tc_tiling_on_sc=True, needs_layout_passes=False),
    )
    per_scene = f(scores, ious)
    return jnp.sum(per_scene[:, 0])


def kernel(logmax_scores, instance_ious, instance_matches):
    del instance_matches
    return _run(logmax_scores, instance_ious)


# speedup vs baseline: 7.8485x; 1.0026x over previous
import functools

import jax
import jax.numpy as jnp
from jax import lax
from jax.experimental import pallas as pl
from jax.experimental.pallas import tpu as pltpu
from jax.experimental.pallas import tpu_sc as plsc

ALPHA = 2.0
NEG_WEIGHT = 1.0
MIN_IOU = 0.05

L = 16
ROWS = 16
B, M, N = 32, 1024, 1024
NCHUNK = M // ROWS
NJC = N // L
NPAIR = NCHUNK // 2


def _body(scores_hbm, ious_hbm, out_hbm,
          iou0, iou1, sc0, sc1, lastrow, colsum, rowvec, out_buf,
          sem_i0, sem_i1, sem_s0, sem_s1):
    cid = lax.axis_index("c")
    sid = lax.axis_index("s")
    b = sid * 2 + cid

    def iou_copy(g, buf, sem):
        return pltpu.make_async_copy(
            ious_hbm.at[b, pl.ds(g * ROWS, ROWS), pl.ds(0, N)], buf, sem)

    def sc_copy(g, buf, sem):
        return pltpu.make_async_copy(
            scores_hbm.at[b, pl.ds(g * ROWS, ROWS)], buf, sem)

    iou_copy(0, iou0, sem_i0).start()
    sc_copy(0, sc0, sem_s0).start()
    iou_copy(1, iou1, sem_i1).start()
    sc_copy(1, sc1, sem_s1).start()

    pltpu.sync_copy(scores_hbm.at[b, pl.ds(M, 1)], lastrow)

    def zbody(j, _):
        colsum[pl.ds(j * L, L)] = jnp.zeros((L,), jnp.float32)
        return 0
    lax.fori_loop(0, NJC, zbody, 0)

    zero_v = jnp.zeros((L,), jnp.float32)
    zero_iv = jnp.zeros((L,), jnp.int32)
    iota16 = jnp.arange(L, dtype=jnp.int32)

    def chunk_compute(g, iou_buf, sc_buf, s1, cnt, n0c, n0d):
        def make_jc_body(base):
            def jc_body(jc, c):
                s1_, cnt_, raccs = c
                off = jc * L
                colacc = colsum[pl.ds(off, L)]
                new_raccs = []
                for i in range(ROWS // 2):
                    vio = iou_buf[base + i, pl.ds(off, L)]
                    vsc = sc_buf[base + i, pl.ds(off, L)]
                    m = vio >= MIN_IOU
                    t = jnp.where(m, jnp.minimum(vio, 1.0), 0.0)
                    s1_ = s1_ + vsc * t
                    cnt_ = cnt_ + jnp.where(m, 1.0, 0.0)
                    colacc = colacc + t
                    new_raccs.append(raccs[i] + t)
                colsum[pl.ds(off, L)] = colacc
                return s1_, cnt_, tuple(new_raccs)
            return jc_body

        for base in (0, ROWS // 2):
            s1, cnt, raccs = lax.fori_loop(
                0, NJC, make_jc_body(base),
                (s1, cnt, (zero_v,) * (ROWS // 2)))
            for i in range(ROWS // 2):
                rowvec[base + i] = raccs[i]

        rowsum = plsc.load_gather(
            rowvec, [iota16, jnp.full((L,), 0, jnp.int32)])
        for c in range(1, L):
            rowsum = rowsum + plsc.load_gather(
                rowvec, [iota16, jnp.full((L,), c, jnp.int32)])
        fvec = jnp.where(rowsum <= 0.001, 1.0, 0.0)
        lc = plsc.load_gather(sc_buf, [iota16, jnp.full((L,), N, jnp.int32)])
        return s1, cnt, n0c + fvec, n0d + fvec * lc

    def pair_body(p, carry):
        s1, cnt, n0c, n0d = carry
        g0 = 2 * p
        iou_copy(g0, iou0, sem_i0).wait()
        sc_copy(g0, sc0, sem_s0).wait()
        s1, cnt, n0c, n0d = chunk_compute(g0, iou0, sc0, s1, cnt, n0c, n0d)
        ge = jnp.minimum(g0 + 2, NCHUNK - 1)
        iou_copy(ge, iou0, sem_i0).start()
        sc_copy(ge, sc0, sem_s0).start()

        g1 = 2 * p + 1
        iou_copy(g1, iou1, sem_i1).wait()
        sc_copy(g1, sc1, sem_s1).wait()
        s1, cnt, n0c, n0d = chunk_compute(g1, iou1, sc1, s1, cnt, n0c, n0d)
        go = jnp.minimum(g1 + 2, NCHUNK - 1)
        iou_copy(go, iou1, sem_i1).start()
        sc_copy(go, sc1, sem_s1).start()
        return s1, cnt, n0c, n0d

    s1, cnt, n0c, n0d = lax.fori_loop(
        0, NPAIR, pair_body, (zero_v, zero_v, zero_v, zero_v))

    iou_copy(NCHUNK - 1, iou0, sem_i0).wait()
    sc_copy(NCHUNK - 1, sc0, sem_s0).wait()
    iou_copy(NCHUNK - 1, iou1, sem_i1).wait()
    sc_copy(NCHUNK - 1, sc1, sem_s1).wait()

    def neg1_body(jc, c):
        n1c_, n1d_ = c
        off = jc * L
        v = colsum[pl.ds(off, L)]
        lr = lastrow[0, pl.ds(off, L)]
        n1 = jnp.where(v <= 0.001, 1.0, 0.0)
        return n1c_ + n1, n1d_ + n1 * lr
    n1c, n1d = lax.fori_loop(0, NJC, neg1_body, (zero_v, zero_v))

    s1s = jnp.full((L,), jnp.sum(s1), jnp.float32)
    cnts = jnp.full((L,), jnp.sum(cnt), jnp.float32)
    n0cs = jnp.full((L,), jnp.sum(n0c), jnp.float32)
    n0ds = jnp.full((L,), jnp.sum(n0d), jnp.float32)
    n1cs = jnp.full((L,), jnp.sum(n1c), jnp.float32)
    n1ds = jnp.full((L,), jnp.sum(n1d), jnp.float32)

    nll_pos = -(ALPHA * s1s) / jnp.maximum(cnts, 1.0)
    nll_neg = (-n0ds - n1ds) / (jnp.maximum(n0cs, 1.0) + jnp.maximum(n1cs, 1.0))
    loss = (nll_pos + NEG_WEIGHT * nll_neg) * (1.0 / B)

    out_buf[...] = loss
    pltpu.sync_copy(out_buf, out_hbm.at[b])


@jax.jit
def _run(scores, ious):
    mesh = plsc.VectorSubcoreMesh(core_axis_name="c", subcore_axis_name="s")
    f = pl.kernel(
        _body,
        out_type=jax.ShapeDtypeStruct((B, L), jnp.float32),
        mesh=mesh,
        scratch_types=[
            pltpu.VMEM((ROWS, N), jnp.float32),
            pltpu.VMEM((ROWS, N), jnp.float32),
            pltpu.VMEM((ROWS, N + 1), jnp.float32),
            pltpu.VMEM((ROWS, N + 1), jnp.float32),
            pltpu.VMEM((1, N + 1), jnp.float32),
            pltpu.VMEM((N,), jnp.float32),
            pltpu.VMEM((ROWS, L), jnp.float32),
            pltpu.VMEM((L,), jnp.float32),
            pltpu.SemaphoreType.DMA,
            pltpu.SemaphoreType.DMA,
            pltpu.SemaphoreType.DMA,
            pltpu.SemaphoreType.DMA,
        ],
        compiler_params=pltpu.CompilerParams(
            use_tc_tiling_on_sc=True, needs_layout_passes=False),
    )
    per_scene = f(scores, ious)
    return jnp.sum(per_scene[:, 0])


def kernel(logmax_scores, instance_ious, instance_matches):
    del instance_matches
    return _run(logmax_scores, instance_ious)
